# baseline (speedup 1.0000x reference)
import functools

import jax
import jax.numpy as jnp
from jax.experimental import pallas as pl
from jax.experimental.pallas import tpu as pltpu

B, F, D = 64, 2048, 128


def _tc_body(nb_ref, rel_ref, a1_ref, a2_ref, fr_ref, fa1_ref, fa2_ref, out_ref):
    b = pl.program_id(0)

    def l2(f_ref, q_ref):
        d = f_ref[0] - q_ref[0]
        return jnp.sum(d * d, axis=1, keepdims=True)

    tot = l2(fr_ref, rel_ref) + l2(fa1_ref, a1_ref) + l2(fa2_ref, a2_ref)
    scores = jnp.exp(-0.5 * tot)
    n = nb_ref[b]
    mask = jax.lax.broadcasted_iota(jnp.int32, (F, 1), 0) < n
    scores = jnp.where(mask, scores, 0.0)
    out_ref[0] = jnp.max(scores, axis=0, keepdims=True)


def kernel(rel, arg1, arg2, facts_rel, facts_arg1, facts_arg2, nb_facts):
    grid_spec = pltpu.PrefetchScalarGridSpec(
        num_scalar_prefetch=1,
        grid=(B,),
        in_specs=[
            pl.BlockSpec((1, 1, D), lambda b, nb: (b, 0, 0)),
            pl.BlockSpec((1, 1, D), lambda b, nb: (b, 0, 0)),
            pl.BlockSpec((1, 1, D), lambda b, nb: (b, 0, 0)),
            pl.BlockSpec((1, F, D), lambda b, nb: (b, 0, 0)),
            pl.BlockSpec((1, F, D), lambda b, nb: (b, 0, 0)),
            pl.BlockSpec((1, F, D), lambda b, nb: (b, 0, 0)),
        ],
        out_specs=pl.BlockSpec((1, 1, 1), lambda b, nb: (b, 0, 0)),
    )
    out = pl.pallas_call(
        _tc_body,
        grid_spec=grid_spec,
        out_shape=jax.ShapeDtypeStruct((B, 1, 1), jnp.float32),
    )(nb_facts, rel.reshape(B, 1, D), arg1.reshape(B, 1, D),
      arg2.reshape(B, 1, D), facts_rel, facts_arg1, facts_arg2)
    return out.reshape(B)
